# Initial kernel scaffold; baseline (speedup 1.0000x reference)
import jax
import jax.numpy as jnp
from jax.experimental import pallas as pl


def kernel(unary, binary, index1, index2):
    raise NotImplementedError("write your pallas kernel here")


# speedup vs baseline: 415.2822x; 415.2822x over previous
import functools

import jax
import jax.numpy as jnp
from jax import lax
from jax.experimental import pallas as pl
from jax.experimental.pallas import tpu as pltpu
from jax.experimental.pallas import tpu_sc as plsc

_E = 320000
_D = 128
_DB = 16
_NN = 10000
_DOUT = 2 * _D + _DB

_CPT = 8
_BE = 1600
_NBLK = _E // _BE
_RB = 1000
_ROWS_PER_W = _E // 32
_NBCHUNK = _ROWS_PER_W // _RB


def _build():
    mesh = plsc.VectorSubcoreMesh(core_axis_name="c", subcore_axis_name="s")

    @functools.partial(
        pl.kernel,
        out_type=jax.ShapeDtypeStruct((_E, _DOUT), jnp.float32),
        mesh=mesh,
        scratch_types=[
            pltpu.VMEM((_NN, _CPT), jnp.float32),
            pltpu.VMEM((_BE, _CPT), jnp.int32),
            pltpu.VMEM((_BE, _CPT), jnp.float32),
            pltpu.VMEM((_RB, _DB), jnp.float32),
        ],
        compiler_params=pltpu.CompilerParams(
            use_tc_tiling_on_sc=False, needs_layout_passes=False),
    )
    def sc_join(unary, binary, idx1, idx2, out, table_v, idx_v, out_v, bin_v):
        c = lax.axis_index("c")
        s = lax.axis_index("s")
        col = s * _CPT

        pltpu.sync_copy(unary.at[pl.ds(0, _NN), pl.ds(col, _CPT)], table_v)

        iota = lax.iota(jnp.int32, 16)
        c_vec = lax.bitwise_and(iota, 7)
        e_vec0 = lax.shift_right_logical(iota, 3)

        def gather_half(idx_hbm, out_col0):
            def blk(b, _):
                e0 = b * _BE
                pltpu.sync_copy(
                    idx_hbm.at[pl.ds(e0, _BE), pl.ds(col, _CPT)], idx_v)

                @plsc.parallel_loop(0, _BE * _CPT // 16, unroll=8)
                def grp(k):
                    e_vec = e_vec0 + 2 * k
                    r = plsc.load_gather(idx_v, [e_vec, c_vec])
                    val = plsc.load_gather(table_v, [r, c_vec])
                    plsc.store_scatter(out_v, [e_vec, c_vec], val)

                pltpu.sync_copy(
                    out_v, out.at[pl.ds(e0, _BE), pl.ds(out_col0 + col, _CPT)])
                return 0

            lax.fori_loop(0, _NBLK, blk, 0)

        @pl.when(c == 0)
        def _():
            gather_half(idx1, 0)

        @pl.when(c == 1)
        def _():
            gather_half(idx2, _D)

        wid = c * 16 + s

        def bchunk(j, _):
            r0 = wid * _ROWS_PER_W + j * _RB
            pltpu.sync_copy(binary.at[pl.ds(r0, _RB), pl.ds(0, _DB)], bin_v)
            pltpu.sync_copy(bin_v, out.at[pl.ds(r0, _RB), pl.ds(2 * _D, _DB)])
            return 0

        lax.fori_loop(0, _NBCHUNK, bchunk, 0)

    return sc_join


_kernel_fn = _build()


def kernel(unary, binary, index1, index2):
    index1 = jnp.squeeze(index1)
    index2 = jnp.squeeze(index2)
    return _kernel_fn(unary, binary, index1, index2)


# speedup vs baseline: 446.6018x; 1.0754x over previous
import functools

import jax
import jax.numpy as jnp
from jax import lax
from jax.experimental import pallas as pl
from jax.experimental.pallas import tpu as pltpu
from jax.experimental.pallas import tpu_sc as plsc

_E = 320000
_D = 128
_DB = 16
_NN = 10000
_DOUT = 2 * _D + _DB

_CPT = 8
_BE = 1600
_NBLK = _E // _BE
_RB = 1000
_ROWS_PER_W = _E // 32
_NBCHUNK = _ROWS_PER_W // _RB


def _build():
    mesh = plsc.VectorSubcoreMesh(core_axis_name="c", subcore_axis_name="s")

    @functools.partial(
        pl.kernel,
        out_type=jax.ShapeDtypeStruct((_DOUT, _E), jnp.float32),
        mesh=mesh,
        scratch_types=[
            pltpu.VMEM((_NN, _CPT), jnp.float32),
            pltpu.VMEM((_BE, _CPT), jnp.int32),
            pltpu.VMEM((_CPT, _BE), jnp.float32),
            pltpu.VMEM((_DB, _RB), jnp.float32),
        ],
        compiler_params=pltpu.CompilerParams(
            use_tc_tiling_on_sc=False, needs_layout_passes=False),
    )
    def sc_join(unary, binary_t, idx1, idx2, out, table_v, idx_v, out_v, bin_v):
        c = lax.axis_index("c")
        s = lax.axis_index("s")
        col = s * _CPT

        pltpu.sync_copy(unary.at[pl.ds(0, _NN), pl.ds(col, _CPT)], table_v)

        iota = lax.iota(jnp.int32, 16)

        def gather_half(idx_hbm, out_row0):
            def blk(b, _):
                e0 = b * _BE
                pltpu.sync_copy(
                    idx_hbm.at[pl.ds(e0, _BE), pl.ds(col, _CPT)], idx_v)

                for j in range(_CPT):
                    j_vec = jnp.full((16,), j, jnp.int32)

                    @plsc.parallel_loop(0, _BE // 16, unroll=8)
                    def grp(k):
                        e_vec = iota + 16 * k
                        r = plsc.load_gather(idx_v, [e_vec, j_vec])
                        val = plsc.load_gather(table_v, [r, j_vec])
                        out_v[j, pl.ds(16 * k, 16)] = val

                pltpu.sync_copy(
                    out_v, out.at[pl.ds(out_row0 + col, _CPT), pl.ds(e0, _BE)])
                return 0

            lax.fori_loop(0, _NBLK, blk, 0)

        @pl.when(c == 0)
        def _():
            gather_half(idx1, 0)

        @pl.when(c == 1)
        def _():
            gather_half(idx2, _D)

        wid = c * 16 + s

        def bchunk(j, _):
            r0 = wid * _ROWS_PER_W + j * _RB
            pltpu.sync_copy(binary_t.at[pl.ds(0, _DB), pl.ds(r0, _RB)], bin_v)
            pltpu.sync_copy(bin_v, out.at[pl.ds(2 * _D, _DB), pl.ds(r0, _RB)])
            return 0

        lax.fori_loop(0, _NBCHUNK, bchunk, 0)

    return sc_join


_kernel_fn = _build()


def kernel(unary, binary, index1, index2):
    index1 = jnp.squeeze(index1)
    index2 = jnp.squeeze(index2)
    out_t = _kernel_fn(unary, binary.T, index1, index2)
    return out_t.T


# speedup vs baseline: 665.1490x; 1.4894x over previous
import functools

import jax
import jax.numpy as jnp
from jax import lax
from jax.experimental import pallas as pl
from jax.experimental.pallas import tpu as pltpu
from jax.experimental.pallas import tpu_sc as plsc

_E = 320000
_D = 128
_DB = 16
_NN = 10000
_DOUT = 2 * _D + _DB

_CPT = 8
_BE = 1600
_NBLK = _E // _BE
_RB = 1000
_ROWS_PER_W = _E // 32
_NBCHUNK = _ROWS_PER_W // _RB


def _build():
    mesh = plsc.VectorSubcoreMesh(core_axis_name="c", subcore_axis_name="s")

    @functools.partial(
        pl.kernel,
        out_type=jax.ShapeDtypeStruct((_DOUT, _E), jnp.float32),
        mesh=mesh,
        scratch_types=[
            pltpu.VMEM((_NN, _CPT), jnp.float32),
            pltpu.VMEM((_BE, _CPT), jnp.int32),
            pltpu.VMEM((_CPT, _BE + 2), jnp.float32),
            pltpu.VMEM((_DB, _RB), jnp.float32),
        ],
        compiler_params=pltpu.CompilerParams(
            use_tc_tiling_on_sc=False, needs_layout_passes=False),
    )
    def sc_join(unary, binary_t, idx1, idx2, out, table_v, idx_v, out_v, bin_v):
        c = lax.axis_index("c")
        s = lax.axis_index("s")
        col = s * _CPT

        pltpu.sync_copy(unary.at[pl.ds(0, _NN), pl.ds(col, _CPT)], table_v)

        iota = lax.iota(jnp.int32, 16)
        c_vec = lax.bitwise_and(iota, 7)
        e_vec0 = lax.shift_right_logical(iota, 3)

        def gather_half(idx_hbm, out_row0):
            def blk(b, _):
                e0 = b * _BE
                pltpu.sync_copy(
                    idx_hbm.at[pl.ds(e0, _BE), pl.ds(col, _CPT)], idx_v)

                @plsc.parallel_loop(0, _BE * _CPT // 16, unroll=8)
                def grp(k):
                    e_vec = e_vec0 + 2 * k
                    r = plsc.load_gather(idx_v, [e_vec, c_vec])
                    val = plsc.load_gather(table_v, [r, c_vec])
                    plsc.store_scatter(out_v, [c_vec, e_vec], val)

                pltpu.sync_copy(
                    out_v.at[pl.ds(0, _CPT), pl.ds(0, _BE)],
                    out.at[pl.ds(out_row0 + col, _CPT), pl.ds(e0, _BE)])
                return 0

            lax.fori_loop(0, _NBLK, blk, 0)

        @pl.when(c == 0)
        def _():
            gather_half(idx1, 0)

        @pl.when(c == 1)
        def _():
            gather_half(idx2, _D)

        wid = c * 16 + s

        def bchunk(j, _):
            r0 = wid * _ROWS_PER_W + j * _RB
            pltpu.sync_copy(binary_t.at[pl.ds(0, _DB), pl.ds(r0, _RB)], bin_v)
            pltpu.sync_copy(bin_v, out.at[pl.ds(2 * _D, _DB), pl.ds(r0, _RB)])
            return 0

        lax.fori_loop(0, _NBCHUNK, bchunk, 0)

    return sc_join


_kernel_fn = _build()


def kernel(unary, binary, index1, index2):
    index1 = jnp.squeeze(index1)
    index2 = jnp.squeeze(index2)
    out_t = _kernel_fn(unary, binary.T, index1, index2)
    return out_t.T


# speedup vs baseline: 775.2282x; 1.1655x over previous
import functools

import jax
import jax.numpy as jnp
from jax import lax
from jax.experimental import pallas as pl
from jax.experimental.pallas import tpu as pltpu
from jax.experimental.pallas import tpu_sc as plsc

_E = 320000
_D = 128
_DB = 16
_NN = 10000
_DOUT = 2 * _D + _DB

_CPT = 8
_BE = 1280
_NBLK = _E // _BE
_BT = _BE // 128
_ETC = _E // 128
_BINT = 4
_NBCH = _ETC // _BINT


def _build():
    mesh = plsc.VectorSubcoreMesh(core_axis_name="c", subcore_axis_name="s")

    @functools.partial(
        pl.kernel,
        out_type=jax.ShapeDtypeStruct((_DOUT // 8, _ETC * 8, 128),
                                      jnp.float32),
        mesh=mesh,
        scratch_types=[
            pltpu.VMEM((_NN, _CPT), jnp.float32),
            pltpu.VMEM((_BE, _CPT), jnp.int32),
            pltpu.VMEM((_CPT, _BE + 2), jnp.float32),
            pltpu.VMEM((1, _BT * 8, 128), jnp.float32),
            pltpu.VMEM((_DB // 2, _BINT * 128), jnp.float32),
            pltpu.VMEM((1, _BINT * 8, 128), jnp.float32),
        ],
        compiler_params=pltpu.CompilerParams(
            use_tc_tiling_on_sc=False, needs_layout_passes=False),
    )
    def sc_join(unary, binary_t, idx1, idx2, out,
                table_v, idx_v, out_v, out3_v, bin_v, bin3_v):
        c = lax.axis_index("c")
        s = lax.axis_index("s")
        col = s * _CPT

        pltpu.sync_copy(unary.at[pl.ds(0, _NN), pl.ds(col, _CPT)], table_v)

        iota = lax.iota(jnp.int32, 16)
        c_vec = lax.bitwise_and(iota, 7)
        e_vec0 = lax.shift_right_logical(iota, 3)
        zero_vec = jnp.zeros((16,), jnp.int32)

        def repack(src_v, dst_v, nrows, ntiles):
            for r in range(nrows):
                @plsc.parallel_loop(0, ntiles * 8, unroll=8)
                def rep(k):
                    val = src_v[r, pl.ds(16 * k, 16)]
                    row_vec = zero_vec + 8 * (k // 8) + r
                    col_vec = iota + (k % 8) * 16
                    plsc.store_scatter(dst_v, [zero_vec, row_vec, col_vec],
                                       val)

        def gather_half(idx_hbm, tr0):
            tr = tr0 + s

            def blk(b, _):
                e0 = b * _BE
                pltpu.sync_copy(
                    idx_hbm.at[pl.ds(e0, _BE), pl.ds(col, _CPT)], idx_v)

                @plsc.parallel_loop(0, _BE * _CPT // 16, unroll=8)
                def grp(k):
                    e_vec = e_vec0 + 2 * k
                    r = plsc.load_gather(idx_v, [e_vec, c_vec])
                    val = plsc.load_gather(table_v, [r, c_vec])
                    plsc.store_scatter(out_v, [c_vec, e_vec], val)

                repack(out_v, out3_v, _CPT, _BT)

                pltpu.sync_copy(
                    out3_v,
                    out.at[pl.ds(tr, 1),
                           pl.ds(pl.multiple_of(b * (_BT * 8), 8), _BT * 8),
                           pl.ds(0, 128)])
                return 0

            lax.fori_loop(0, _NBLK, blk, 0)

        @pl.when(c == 0)
        def _():
            gather_half(idx1, 0)

        @pl.when(c == 1)
        def _():
            gather_half(idx2, 16)

        def bloop(m, _):
            ch = m * 16 + s

            @pl.when(ch < _NBCH)
            def _():
                e0 = ch * (_BINT * 128)
                pltpu.sync_copy(
                    binary_t.at[pl.ds(c * 8, 8), pl.ds(e0, _BINT * 128)],
                    bin_v)
                repack(bin_v, bin3_v, 8, _BINT)
                pltpu.sync_copy(
                    bin3_v,
                    out.at[pl.ds(32 + c, 1),
                           pl.ds(pl.multiple_of(ch * (_BINT * 8), 8),
                                 _BINT * 8),
                           pl.ds(0, 128)])
            return 0

        lax.fori_loop(0, (_NBCH + 15) // 16, bloop, 0)

    return sc_join


_kernel_fn = _build()


def kernel(unary, binary, index1, index2):
    index1 = jnp.squeeze(index1)
    index2 = jnp.squeeze(index2)
    out3 = _kernel_fn(unary, binary.T, index1, index2)
    out4 = out3.reshape(_DOUT // 8, _ETC, 8, 128)
    out_t = out4.transpose(0, 2, 1, 3).reshape(_DOUT, _E)
    return out_t.T


# speedup vs baseline: 958.6559x; 1.2366x over previous
import functools

import jax
import jax.numpy as jnp
from jax import lax
from jax.experimental import pallas as pl
from jax.experimental.pallas import tpu as pltpu
from jax.experimental.pallas import tpu_sc as plsc

_E = 320000
_D = 128
_DB = 16
_NN = 10000
_DOUT = 2 * _D + _DB

_CPT = 8
_BE = 640
_NBLK = _E // _BE
_BT = _BE // 128
_ETC = _E // 128
_BINT = 4
_NBCH = _ETC // _BINT


def _build():
    mesh = plsc.VectorSubcoreMesh(core_axis_name="c", subcore_axis_name="s")

    @functools.partial(
        pl.kernel,
        out_type=jax.ShapeDtypeStruct((_DOUT // 8, _ETC * 8, 128),
                                      jnp.float32),
        mesh=mesh,
        scratch_types=[
            pltpu.VMEM((_NN, _CPT), jnp.float32),
            pltpu.VMEM((_BE, _CPT), jnp.int32),
            pltpu.VMEM((_BE, _CPT), jnp.int32),
            pltpu.VMEM((_CPT, _BE + 2), jnp.float32),
            pltpu.VMEM((1, _BT * 8, 128), jnp.float32),
            pltpu.VMEM((1, _BT * 8, 128), jnp.float32),
            pltpu.VMEM((_DB // 2, _BINT * 128), jnp.float32),
            pltpu.VMEM((1, _BINT * 8, 128), jnp.float32),
            pltpu.SemaphoreType.DMA,
            pltpu.SemaphoreType.DMA,
            pltpu.SemaphoreType.DMA,
            pltpu.SemaphoreType.DMA,
        ],
        compiler_params=pltpu.CompilerParams(
            use_tc_tiling_on_sc=False, needs_layout_passes=False),
    )
    def sc_join(unary, binary_t, idx1, idx2, out,
                table_v, idx_v0, idx_v1, out_v, out3_v0, out3_v1,
                bin_v, bin3_v, sem_i0, sem_i1, sem_o0, sem_o1):
        c = lax.axis_index("c")
        s = lax.axis_index("s")
        col = s * _CPT

        pltpu.sync_copy(unary.at[pl.ds(0, _NN), pl.ds(col, _CPT)], table_v)

        iota = lax.iota(jnp.int32, 16)
        c_vec = lax.bitwise_and(iota, 7)
        e_vec0 = lax.shift_right_logical(iota, 3)
        zero_vec = jnp.zeros((16,), jnp.int32)

        idx_bufs = (idx_v0, idx_v1)
        out3_bufs = (out3_v0, out3_v1)
        sem_i = (sem_i0, sem_i1)
        sem_o = (sem_o0, sem_o1)

        def repack(src_v, dst_v, nrows, ntiles):
            for r in range(nrows):
                @plsc.parallel_loop(0, ntiles * 8, unroll=8)
                def rep(k):
                    val = src_v[r, pl.ds(16 * k, 16)]
                    row_vec = zero_vec + 8 * (k // 8) + r
                    col_vec = iota + (k % 8) * 16
                    plsc.store_scatter(dst_v, [zero_vec, row_vec, col_vec],
                                       val)

        def gather_half(idx_hbm, tr0):
            tr = tr0 + s

            def idx_src(b):
                return idx_hbm.at[pl.ds(b * _BE, _BE), pl.ds(col, _CPT)]

            def out_dst(b):
                return out.at[pl.ds(tr, 1),
                              pl.ds(pl.multiple_of(b * (_BT * 8), 8),
                                    _BT * 8),
                              pl.ds(0, 128)]

            pltpu.async_copy(idx_src(0), idx_bufs[0], sem_i[0])

            def pair(b2, _):
                for p in range(2):
                    b = 2 * b2 + p
                    pltpu.make_async_copy(
                        idx_src(b), idx_bufs[p], sem_i[p]).wait()

                    @pl.when(b + 1 < _NBLK)
                    def _():
                        pltpu.async_copy(
                            idx_src(b + 1), idx_bufs[1 - p], sem_i[1 - p])

                    @plsc.parallel_loop(0, _BE * _CPT // 16, unroll=8)
                    def grp(k):
                        e_vec = e_vec0 + 2 * k
                        r = plsc.load_gather(idx_bufs[p], [e_vec, c_vec])
                        val = plsc.load_gather(table_v, [r, c_vec])
                        plsc.store_scatter(out_v, [c_vec, e_vec], val)

                    @pl.when(b2 > 0)
                    def _():
                        pltpu.make_async_copy(
                            out3_bufs[p], out_dst(b - 2), sem_o[p]).wait()

                    repack(out_v, out3_bufs[p], _CPT, _BT)

                    pltpu.async_copy(out3_bufs[p], out_dst(b), sem_o[p])
                return 0

            lax.fori_loop(0, _NBLK // 2, pair, 0)

            for p in range(2):
                pltpu.make_async_copy(
                    out3_bufs[p], out_dst(_NBLK - 2 + p), sem_o[p]).wait()

        @pl.when(c == 0)
        def _():
            gather_half(idx1, 0)

        @pl.when(c == 1)
        def _():
            gather_half(idx2, 16)

        def bloop(m, _):
            ch = m * 16 + s

            @pl.when(ch < _NBCH)
            def _():
                e0 = ch * (_BINT * 128)
                pltpu.sync_copy(
                    binary_t.at[pl.ds(c * 8, 8), pl.ds(e0, _BINT * 128)],
                    bin_v)
                repack(bin_v, bin3_v, 8, _BINT)
                pltpu.sync_copy(
                    bin3_v,
                    out.at[pl.ds(32 + c, 1),
                           pl.ds(pl.multiple_of(ch * (_BINT * 8), 8),
                                 _BINT * 8),
                           pl.ds(0, 128)])
            return 0

        lax.fori_loop(0, (_NBCH + 15) // 16, bloop, 0)

    return sc_join


_kernel_fn = _build()


def kernel(unary, binary, index1, index2):
    index1 = jnp.squeeze(index1)
    index2 = jnp.squeeze(index2)
    out3 = _kernel_fn(unary, binary.T, index1, index2)
    out4 = out3.reshape(_DOUT // 8, _ETC, 8, 128)
    out_t = out4.transpose(0, 2, 1, 3).reshape(_DOUT, _E)
    return out_t.T


# speedup vs baseline: 959.0709x; 1.0004x over previous
import functools

import jax
import jax.numpy as jnp
from jax import lax
from jax.experimental import pallas as pl
from jax.experimental.pallas import tpu as pltpu
from jax.experimental.pallas import tpu_sc as plsc

_E = 320000
_D = 128
_DB = 16
_NN = 10000
_DOUT = 2 * _D + _DB

_CPT = 8
_BE = 640
_NBLK = _E // _BE
_BT = _BE // 128
_ETC = _E // 128
_BINT = 4
_NBCH = _ETC // _BINT


def _build():
    mesh = plsc.VectorSubcoreMesh(core_axis_name="c", subcore_axis_name="s")

    @functools.partial(
        pl.kernel,
        out_type=jax.ShapeDtypeStruct((_DOUT // 8, _ETC * 8, 128),
                                      jnp.float32),
        mesh=mesh,
        scratch_types=[
            pltpu.VMEM((_NN, _CPT), jnp.float32),
            pltpu.VMEM((_BE, _CPT), jnp.int32),
            pltpu.VMEM((_BE, _CPT), jnp.int32),
            pltpu.VMEM((_CPT, _BE + 2), jnp.float32),
            pltpu.VMEM((1, _BT * 8, 128), jnp.float32),
            pltpu.VMEM((1, _BT * 8, 128), jnp.float32),
            pltpu.VMEM((_DB // 2, _BINT * 128), jnp.float32),
            pltpu.VMEM((1, _BINT * 8, 128), jnp.float32),
            pltpu.SemaphoreType.DMA,
            pltpu.SemaphoreType.DMA,
            pltpu.SemaphoreType.DMA,
            pltpu.SemaphoreType.DMA,
        ],
        compiler_params=pltpu.CompilerParams(
            use_tc_tiling_on_sc=False, needs_layout_passes=False),
    )
    def sc_join(unary, binary_t, idx1, idx2, out,
                table_v, idx_v0, idx_v1, out_v, out3_v0, out3_v1,
                bin_v, bin3_v, sem_i0, sem_i1, sem_o0, sem_o1):
        c = lax.axis_index("c")
        s = lax.axis_index("s")
        col = s * _CPT

        pltpu.sync_copy(unary.at[pl.ds(0, _NN), pl.ds(col, _CPT)], table_v)

        iota = lax.iota(jnp.int32, 16)
        c_vec = lax.bitwise_and(iota, 7)
        e_vec0 = lax.shift_right_logical(iota, 3)
        zero_vec = jnp.zeros((16,), jnp.int32)

        idx_bufs = (idx_v0, idx_v1)
        out3_bufs = (out3_v0, out3_v1)
        sem_i = (sem_i0, sem_i1)
        sem_o = (sem_o0, sem_o1)

        def repack(src_v, dst_v, nrows, ntiles):
            for r in range(nrows):
                @plsc.parallel_loop(0, ntiles * 8, unroll=8)
                def rep(k):
                    val = src_v[r, pl.ds(16 * k, 16)]
                    row_vec = zero_vec + 8 * (k // 8) + r
                    col_vec = iota + (k % 8) * 16
                    plsc.store_scatter(dst_v, [zero_vec, row_vec, col_vec],
                                       val)

        def gather_half(idx_hbm, tr0):
            tr = tr0 + s

            def idx_src(b):
                return idx_hbm.at[pl.ds(b * _BE, _BE), pl.ds(col, _CPT)]

            def out_dst(b):
                return out.at[pl.ds(tr, 1),
                              pl.ds(pl.multiple_of(b * (_BT * 8), 8),
                                    _BT * 8),
                              pl.ds(0, 128)]

            pltpu.async_copy(idx_src(0), idx_bufs[0], sem_i[0])

            def pair(b2, _):
                for p in range(2):
                    b = 2 * b2 + p
                    pltpu.make_async_copy(
                        idx_src(b), idx_bufs[p], sem_i[p]).wait()

                    @pl.when(b + 1 < _NBLK)
                    def _():
                        pltpu.async_copy(
                            idx_src(b + 1), idx_bufs[1 - p], sem_i[1 - p])

                    @plsc.parallel_loop(0, _BE * _CPT // 16, unroll=16)
                    def grp(k):
                        e_vec = e_vec0 + 2 * k
                        r = plsc.load_gather(idx_bufs[p], [e_vec, c_vec])
                        val = plsc.load_gather(table_v, [r, c_vec])
                        plsc.store_scatter(out_v, [c_vec, e_vec], val)

                    @pl.when(b2 > 0)
                    def _():
                        pltpu.make_async_copy(
                            out3_bufs[p], out_dst(b - 2), sem_o[p]).wait()

                    repack(out_v, out3_bufs[p], _CPT, _BT)

                    pltpu.async_copy(out3_bufs[p], out_dst(b), sem_o[p])
                return 0

            lax.fori_loop(0, _NBLK // 2, pair, 0)

            for p in range(2):
                pltpu.make_async_copy(
                    out3_bufs[p], out_dst(_NBLK - 2 + p), sem_o[p]).wait()

        @pl.when(c == 0)
        def _():
            gather_half(idx1, 0)

        @pl.when(c == 1)
        def _():
            gather_half(idx2, 16)

        def bloop(m, _):
            ch = m * 16 + s

            @pl.when(ch < _NBCH)
            def _():
                e0 = ch * (_BINT * 128)
                pltpu.sync_copy(
                    binary_t.at[pl.ds(c * 8, 8), pl.ds(e0, _BINT * 128)],
                    bin_v)
                repack(bin_v, bin3_v, 8, _BINT)
                pltpu.sync_copy(
                    bin3_v,
                    out.at[pl.ds(32 + c, 1),
                           pl.ds(pl.multiple_of(ch * (_BINT * 8), 8),
                                 _BINT * 8),
                           pl.ds(0, 128)])
            return 0

        lax.fori_loop(0, (_NBCH + 15) // 16, bloop, 0)

    return sc_join


_kernel_fn = _build()


def kernel(unary, binary, index1, index2):
    index1 = jnp.squeeze(index1)
    index2 = jnp.squeeze(index2)
    out3 = _kernel_fn(unary, binary.T, index1, index2)
    out4 = out3.reshape(_DOUT // 8, _ETC, 8, 128)
    out_t = out4.transpose(0, 2, 1, 3).reshape(_DOUT, _E)
    return out_t.T


# speedup vs baseline: 959.1770x; 1.0001x over previous
import functools

import jax
import jax.numpy as jnp
from jax import lax
from jax.experimental import pallas as pl
from jax.experimental.pallas import tpu as pltpu
from jax.experimental.pallas import tpu_sc as plsc

_E = 320000
_D = 128
_DB = 16
_NN = 10000
_DOUT = 2 * _D + _DB

_CPT = 8
_BE = 640
_NBLK = _E // _BE
_BT = _BE // 128
_ETC = _E // 128
_BINT = 4
_NBCH = _ETC // _BINT


def _build():
    mesh = plsc.VectorSubcoreMesh(core_axis_name="c", subcore_axis_name="s")

    @functools.partial(
        pl.kernel,
        out_type=jax.ShapeDtypeStruct((_DOUT // 8, _ETC * 8, 128),
                                      jnp.float32),
        mesh=mesh,
        scratch_types=[
            pltpu.VMEM((_NN, _CPT), jnp.float32),
            pltpu.VMEM((_BE, _CPT), jnp.int32),
            pltpu.VMEM((_BE, _CPT), jnp.int32),
            pltpu.VMEM((1, _BT * 8, 130), jnp.float32),
            pltpu.VMEM((1, _BT * 8, 130), jnp.float32),
            pltpu.VMEM((_DB // 2, _BINT * 128), jnp.float32),
            pltpu.VMEM((1, _BINT * 8, 128), jnp.float32),
            pltpu.SemaphoreType.DMA,
            pltpu.SemaphoreType.DMA,
            pltpu.SemaphoreType.DMA,
            pltpu.SemaphoreType.DMA,
        ],
        compiler_params=pltpu.CompilerParams(
            use_tc_tiling_on_sc=False, needs_layout_passes=False),
    )
    def sc_join(unary, binary_t, idx1, idx2, out,
                table_v, idx_v0, idx_v1, out3_v0, out3_v1,
                bin_v, bin3_v, sem_i0, sem_i1, sem_o0, sem_o1):
        c = lax.axis_index("c")
        s = lax.axis_index("s")
        col = s * _CPT

        pltpu.sync_copy(unary.at[pl.ds(0, _NN), pl.ds(col, _CPT)], table_v)

        iota = lax.iota(jnp.int32, 16)
        c_vec = lax.bitwise_and(iota, 7)
        e_vec0 = lax.shift_right_logical(iota, 3)
        zero_vec = jnp.zeros((16,), jnp.int32)

        idx_bufs = (idx_v0, idx_v1)
        out3_bufs = (out3_v0, out3_v1)
        sem_i = (sem_i0, sem_i1)
        sem_o = (sem_o0, sem_o1)

        def repack(src_v, dst_v, nrows, ntiles):
            for r in range(nrows):
                @plsc.parallel_loop(0, ntiles * 8, unroll=8)
                def rep(k):
                    val = src_v[r, pl.ds(16 * k, 16)]
                    row_vec = zero_vec + 8 * (k // 8) + r
                    col_vec = iota + (k % 8) * 16
                    plsc.store_scatter(dst_v, [zero_vec, row_vec, col_vec],
                                       val)

        def gather_half(idx_hbm, tr0):
            tr = tr0 + s

            def idx_src(b):
                return idx_hbm.at[pl.ds(b * _BE, _BE), pl.ds(col, _CPT)]

            def out_dst(b):
                return out.at[pl.ds(tr, 1),
                              pl.ds(pl.multiple_of(b * (_BT * 8), 8),
                                    _BT * 8),
                              pl.ds(0, 128)]

            def out3_src(p):
                return out3_bufs[p].at[pl.ds(0, 1), pl.ds(0, _BT * 8),
                                       pl.ds(0, 128)]

            pltpu.async_copy(idx_src(0), idx_bufs[0], sem_i[0])

            def pair(b2, _):
                for p in range(2):
                    b = 2 * b2 + p
                    pltpu.make_async_copy(
                        idx_src(b), idx_bufs[p], sem_i[p]).wait()

                    @pl.when(b + 1 < _NBLK)
                    def _():
                        pltpu.async_copy(
                            idx_src(b + 1), idx_bufs[1 - p], sem_i[1 - p])

                    @pl.when(b2 > 0)
                    def _():
                        pltpu.make_async_copy(
                            out3_src(p), out_dst(b - 2), sem_o[p]).wait()

                    @plsc.parallel_loop(0, _BE * _CPT // 16, unroll=16)
                    def grp(k):
                        e_vec = e_vec0 + 2 * k
                        tcb = k // 64
                        r = plsc.load_gather(idx_bufs[p], [e_vec, c_vec])
                        val = plsc.load_gather(table_v, [r, c_vec])
                        plsc.store_scatter(
                            out3_bufs[p],
                            [zero_vec, 8 * tcb + c_vec, e_vec - 128 * tcb],
                            val)

                    pltpu.async_copy(out3_src(p), out_dst(b), sem_o[p])
                return 0

            lax.fori_loop(0, _NBLK // 2, pair, 0)

            for p in range(2):
                pltpu.make_async_copy(
                    out3_src(p), out_dst(_NBLK - 2 + p), sem_o[p]).wait()

        @pl.when(c == 0)
        def _():
            gather_half(idx1, 0)

        @pl.when(c == 1)
        def _():
            gather_half(idx2, 16)

        def bloop(m, _):
            ch = m * 16 + s

            @pl.when(ch < _NBCH)
            def _():
                e0 = ch * (_BINT * 128)
                pltpu.sync_copy(
                    binary_t.at[pl.ds(c * 8, 8), pl.ds(e0, _BINT * 128)],
                    bin_v)
                repack(bin_v, bin3_v, 8, _BINT)
                pltpu.sync_copy(
                    bin3_v,
                    out.at[pl.ds(32 + c, 1),
                           pl.ds(pl.multiple_of(ch * (_BINT * 8), 8),
                                 _BINT * 8),
                           pl.ds(0, 128)])
            return 0

        lax.fori_loop(0, (_NBCH + 15) // 16, bloop, 0)

    return sc_join


_kernel_fn = _build()


def kernel(unary, binary, index1, index2):
    index1 = jnp.squeeze(index1)
    index2 = jnp.squeeze(index2)
    out3 = _kernel_fn(unary, binary.T, index1, index2)
    out4 = out3.reshape(_DOUT // 8, _ETC, 8, 128)
    out_t = out4.transpose(0, 2, 1, 3).reshape(_DOUT, _E)
    return out_t.T
